# Initial kernel scaffold; baseline (speedup 1.0000x reference)
import jax
import jax.numpy as jnp
from jax.experimental import pallas as pl


def kernel(data, yr_real, yr_imag, mt_real, mt_imag, x_real, x_imag, y_real, y_imag, m_real, m_imag, d_real, d_imag, t_real, t_imag):
    raise NotImplementedError("write your pallas kernel here")


# speedup vs baseline: 1.7775x; 1.7775x over previous
import functools

import jax
import jax.numpy as jnp
from jax import lax
from jax.experimental import pallas as pl
from jax.experimental.pallas import tpu as pltpu
from jax.experimental.pallas import tpu_sc as plsc

VOCAB = 100001
FEAT = 64
NFIELD = 7
B, T = 4096, 50
N = B * T
NC, NS = 2, 16
NW = NC * NS
TOK_PER_W = N // NW
CHUNK = 128
NCHUNK = TOK_PER_W // CHUNK


def _sc_body(idx_hbm, *refs):
    tables = refs[:2 * NFIELD]
    outs = refs[2 * NFIELD:4 * NFIELD]
    idx_v, rbuf, ibuf, sem_r, sem_i = refs[4 * NFIELD:]

    wid = lax.axis_index("s") * NC + lax.axis_index("c")
    tile_base = wid * TOK_PER_W

    def chunk_body(c, carry):
        base = tile_base + c * CHUNK
        pltpu.sync_copy(idx_hbm.at[:, pl.ds(base, CHUNK)], idx_v)
        for f in range(NFIELD):
            cp_r = pltpu.async_copy(tables[2 * f].at[idx_v.at[f]], rbuf, sem_r)
            cp_i = pltpu.async_copy(tables[2 * f + 1].at[idx_v.at[f]], ibuf, sem_i)
            cp_r.wait()
            pltpu.sync_copy(rbuf, outs[2 * f].at[pl.ds(base, CHUNK), :])
            cp_i.wait()
            pltpu.sync_copy(ibuf, outs[2 * f + 1].at[pl.ds(base, CHUNK), :])
        return carry

    lax.fori_loop(0, NCHUNK, chunk_body, 0)


@jax.jit
def _sc_gather(idxT, *tables):
    fn = pl.kernel(
        _sc_body,
        out_type=tuple(
            jax.ShapeDtypeStruct((N, FEAT), jnp.float32)
            for _ in range(2 * NFIELD)
        ),
        mesh=plsc.VectorSubcoreMesh(core_axis_name="c", subcore_axis_name="s"),
        scratch_types=[
            pltpu.VMEM((NFIELD, CHUNK), jnp.int32),
            pltpu.VMEM((CHUNK, FEAT), jnp.float32),
            pltpu.VMEM((CHUNK, FEAT), jnp.float32),
            pltpu.SemaphoreType.DMA,
            pltpu.SemaphoreType.DMA,
        ],
        compiler_params=pltpu.CompilerParams(use_tc_tiling_on_sc=False),
    )
    return fn(idxT, *tables)


def kernel(data, yr_real, yr_imag, mt_real, mt_imag, x_real, x_imag,
           y_real, y_imag, m_real, m_imag, d_real, d_imag, t_real, t_imag):
    idxT = data.reshape(N, NFIELD).T
    outs = _sc_gather(idxT, yr_real, yr_imag, mt_real, mt_imag, x_real,
                      x_imag, y_real, y_imag, m_real, m_imag, d_real,
                      d_imag, t_real, t_imag)
    embs = [lax.complex(outs[2 * f], outs[2 * f + 1]) for f in range(NFIELD)]
    return jnp.concatenate(embs, axis=-1).reshape(B, T, NFIELD * FEAT)


# speedup vs baseline: 2.1595x; 1.2149x over previous
import jax
import jax.numpy as jnp
from jax import lax
from jax.experimental import pallas as pl
from jax.experimental.pallas import tpu as pltpu
from jax.experimental.pallas import tpu_sc as plsc

VOCAB = 100001
FEAT = 64
NFIELD = 7
B, T = 4096, 50
N = B * T
NC, NS = 2, 16
NW = NC * NS
TOK_PER_W = N // NW
CHUNK = 128
NCHUNK = TOK_PER_W // CHUNK


def _sc_body(idx_hbm, *refs):
    tables = refs[:2 * NFIELD]
    re_out, im_out = refs[2 * NFIELD:2 * NFIELD + 2]
    idx_v, rbuf, ibuf, sem_r, sem_i = refs[2 * NFIELD + 2:]

    wid = lax.axis_index("s") * NC + lax.axis_index("c")
    tile_base = wid * TOK_PER_W

    def chunk_body(c, carry):
        base = tile_base + c * CHUNK
        pltpu.sync_copy(idx_hbm.at[:, pl.ds(base, CHUNK)], idx_v)
        for f in range(NFIELD):
            cp_r = pltpu.async_copy(tables[2 * f].at[idx_v.at[f]], rbuf, sem_r)
            cp_i = pltpu.async_copy(tables[2 * f + 1].at[idx_v.at[f]], ibuf, sem_i)
            cp_r.wait()
            pltpu.sync_copy(rbuf, re_out.at[pl.ds(base, CHUNK), pl.ds(f * FEAT, FEAT)])
            cp_i.wait()
            pltpu.sync_copy(ibuf, im_out.at[pl.ds(base, CHUNK), pl.ds(f * FEAT, FEAT)])
        return carry

    lax.fori_loop(0, NCHUNK, chunk_body, 0)


@jax.jit
def _sc_gather(idxT, *tables):
    fn = pl.kernel(
        _sc_body,
        out_type=(
            jax.ShapeDtypeStruct((N, NFIELD * FEAT), jnp.float32),
            jax.ShapeDtypeStruct((N, NFIELD * FEAT), jnp.float32),
        ),
        mesh=plsc.VectorSubcoreMesh(core_axis_name="c", subcore_axis_name="s"),
        scratch_types=[
            pltpu.VMEM((NFIELD, CHUNK), jnp.int32),
            pltpu.VMEM((CHUNK, FEAT), jnp.float32),
            pltpu.VMEM((CHUNK, FEAT), jnp.float32),
            pltpu.SemaphoreType.DMA,
            pltpu.SemaphoreType.DMA,
        ],
        compiler_params=pltpu.CompilerParams(use_tc_tiling_on_sc=False),
    )
    return fn(idxT, *tables)


def kernel(data, yr_real, yr_imag, mt_real, mt_imag, x_real, x_imag,
           y_real, y_imag, m_real, m_imag, d_real, d_imag, t_real, t_imag):
    idxT = data.reshape(N, NFIELD).T
    re, im = _sc_gather(idxT, yr_real, yr_imag, mt_real, mt_imag, x_real,
                        x_imag, y_real, y_imag, m_real, m_imag, d_real,
                        d_imag, t_real, t_imag)
    return lax.complex(re, im).reshape(B, T, NFIELD * FEAT)
